# baseline (speedup 1.0000x reference)
import functools

import jax
import jax.numpy as jnp
from jax import lax
from jax.experimental import pallas as pl
from jax.experimental.pallas import tpu as pltpu, tpu_sc as plsc


def _make_gather(n_out: int, d: int):
    info = plsc.get_sparse_core_info()
    nc, ns, nl = info.num_cores, info.num_subcores, info.num_lanes
    nw = nc * ns
    assert n_out % nw == 0
    per_w = n_out // nw
    chunk = 8
    nbuf = 6
    la = 3
    n_chunks = per_w // chunk
    mesh = plsc.VectorSubcoreMesh(core_axis_name="c", subcore_axis_name="s")

    @functools.partial(
        pl.kernel,
        mesh=mesh,
        out_type=jax.ShapeDtypeStruct((n_out, 2 * d), jnp.float32),
        scratch_types=[
            pltpu.VMEM((2 * per_w,), jnp.int32),
            pltpu.VMEM((per_w,), jnp.int32),
            pltpu.VMEM((per_w,), jnp.int32),
            pltpu.VMEM((nbuf, chunk, d), jnp.float32),
            pltpu.VMEM((nbuf, chunk, d), jnp.float32),
            pltpu.SemaphoreType.DMA,
            pltpu.SemaphoreType.DMA,
        ],
    )
    def gather_k(table_hbm, idx_hbm, out_hbm,
                 idx_v, idx_ev, idx_ov, buf_e, buf_o, gsem, wsem):
        wid = lax.axis_index("s") * nc + lax.axis_index("c")
        base = wid * per_w
        pltpu.sync_copy(idx_hbm.at[pl.ds(2 * base, 2 * per_w)], idx_v)

        lanes = lax.iota(jnp.int32, nl)
        g_e = (2 * lanes) % nl
        g_o = (2 * lanes + 1) % nl
        lo = lanes < (nl // 2)
        for j in range(per_w // nl):
            a = idx_v[pl.ds(2 * nl * j, nl)]
            b = idx_v[pl.ds(2 * nl * j + nl, nl)]
            idx_ev[pl.ds(nl * j, nl)] = jnp.where(
                lo,
                a.at[g_e].get(mode="promise_in_bounds"),
                b.at[g_e].get(mode="promise_in_bounds"))
            idx_ov[pl.ds(nl * j, nl)] = jnp.where(
                lo,
                a.at[g_o].get(mode="promise_in_bounds"),
                b.at[g_o].get(mode="promise_in_bounds"))

        def start_gathers(j):
            b = j % nbuf
            ge = pltpu.async_copy(
                table_hbm.at[idx_ev.at[pl.ds(j * chunk, chunk)]],
                buf_e.at[b], gsem)
            go = pltpu.async_copy(
                table_hbm.at[idx_ov.at[pl.ds(j * chunk, chunk)]],
                buf_o.at[b], gsem)
            return ge, go

        def start_writes(j):
            b = j % nbuf
            row = base + j * chunk
            we = pltpu.async_copy(
                buf_e.at[b], out_hbm.at[pl.ds(row, chunk), pl.ds(0, d)], wsem)
            wo = pltpu.async_copy(
                buf_o.at[b], out_hbm.at[pl.ds(row, chunk), pl.ds(d, d)], wsem)
            return we, wo

        gathers = [None] * n_chunks
        writes = [None] * n_chunks
        waited = set()
        for j in range(min(la, n_chunks)):
            gathers[j] = start_gathers(j)
        for i in range(n_chunks):
            j = i + la
            if j < n_chunks:
                k = j - nbuf
                if k >= 0:
                    for w in writes[k]:
                        w.wait()
                    waited.add(k)
                gathers[j] = start_gathers(j)
            for g in gathers[i]:
                g.wait()
            writes[i] = start_writes(i)
        for i in range(n_chunks):
            if i not in waited:
                for w in writes[i]:
                    w.wait()

    return gather_k


def kernel(hidden, word_absolute_position):
    B, S, D = hidden.shape
    table = hidden.reshape(B * S, D)
    idx = word_absolute_position.astype(jnp.int32)
    n_idx = idx.shape[0]
    return _make_gather(n_idx // 2, D)(table, idx)


# speedup vs baseline: 1.0115x; 1.0115x over previous
import functools

import jax
import jax.numpy as jnp
from jax import lax
from jax.experimental import pallas as pl
from jax.experimental.pallas import tpu as pltpu, tpu_sc as plsc


def _make_gather(n_out: int, d: int):
    info = plsc.get_sparse_core_info()
    nc, ns, nl = info.num_cores, info.num_subcores, info.num_lanes
    nw = nc * ns
    assert n_out % nw == 0
    per_w = n_out // nw
    chunk = 16
    nbuf = 3
    n_chunks = per_w // chunk
    mesh = plsc.VectorSubcoreMesh(core_axis_name="c", subcore_axis_name="s")

    @functools.partial(
        pl.kernel,
        mesh=mesh,
        out_type=jax.ShapeDtypeStruct((n_out, 2 * d), jnp.float32),
        scratch_types=[
            pltpu.VMEM((2 * per_w,), jnp.int32),
            pltpu.VMEM((2 * per_w,), jnp.int32),
            pltpu.VMEM((nbuf, 2 * chunk, d), jnp.float32),
            pltpu.SemaphoreType.DMA,
            pltpu.SemaphoreType.DMA,
        ],
    )
    def gather_k(table_hbm, idx_hbm, out_hbm,
                 idx_v, idx_g, rows_v, gsem, wsem):
        wid = lax.axis_index("s") * nc + lax.axis_index("c")
        base = wid * per_w
        pltpu.sync_copy(idx_hbm.at[pl.ds(2 * base, 2 * per_w)], idx_v)

        lanes = lax.iota(jnp.int32, nl)
        g_e = (2 * lanes) % nl
        g_o = (2 * lanes + 1) % nl
        lo = lanes < (nl // 2)
        for j in range(2 * per_w // (2 * nl)):
            a = idx_v[pl.ds(2 * nl * j, nl)]
            b = idx_v[pl.ds(2 * nl * j + nl, nl)]
            idx_g[pl.ds(2 * nl * j, nl)] = jnp.where(
                lo,
                a.at[g_e].get(mode="promise_in_bounds"),
                b.at[g_e].get(mode="promise_in_bounds"))
            idx_g[pl.ds(2 * nl * j + nl, nl)] = jnp.where(
                lo,
                a.at[g_o].get(mode="promise_in_bounds"),
                b.at[g_o].get(mode="promise_in_bounds"))

        def start_gather(j):
            return pltpu.async_copy(
                table_hbm.at[idx_g.at[pl.ds(j * 2 * chunk, 2 * chunk)]],
                rows_v.at[j % nbuf], gsem)

        def start_writes(j):
            b = j % nbuf
            row = base + j * chunk
            we = pltpu.async_copy(
                rows_v.at[b, pl.ds(0, chunk)],
                out_hbm.at[pl.ds(row, chunk), pl.ds(0, d)], wsem)
            wo = pltpu.async_copy(
                rows_v.at[b, pl.ds(chunk, chunk)],
                out_hbm.at[pl.ds(row, chunk), pl.ds(d, d)], wsem)
            return we, wo

        gathers = [None] * n_chunks
        writes = [None] * n_chunks
        for j in range(min(nbuf, n_chunks)):
            gathers[j] = start_gather(j)
        for i in range(n_chunks):
            gathers[i].wait()
            writes[i] = start_writes(i)
            nxt = i + nbuf
            if nxt < n_chunks:
                for w in writes[i]:
                    w.wait()
                gathers[nxt] = start_gather(nxt)
        for i in range(max(0, n_chunks - nbuf), n_chunks):
            for w in writes[i]:
                w.wait()

    return gather_k


def kernel(hidden, word_absolute_position):
    B, S, D = hidden.shape
    table = hidden.reshape(B * S, D)
    idx = word_absolute_position.astype(jnp.int32)
    n_idx = idx.shape[0]
    return _make_gather(n_idx // 2, D)(table, idx)
